# baseline (speedup 1.0000x reference)
import functools

import jax
import jax.numpy as jnp
from jax import lax
from jax.experimental import pallas as pl
from jax.experimental.pallas import tpu as pltpu
from jax.experimental.pallas import tpu_sc as plsc

_V = 1000000
_B = 16384
_F = 26
_K = 5
_T = 2 * _K

_NC = 2
_NS = 16
_NW = _NC * _NS
_BPW = _B // _NW
_CB = 64
_NCH = _BPW // _CB

_mesh = plsc.VectorSubcoreMesh(core_axis_name="c", subcore_axis_name="s")


@functools.partial(
    pl.kernel,
    out_type=jax.ShapeDtypeStruct((_NW, _T, _BPW), jnp.float32),
    mesh=_mesh,
    scratch_types=[
        pltpu.VMEM((_F, _BPW), jnp.int32),
        pltpu.VMEM((2, _T, _CB), jnp.float32),
        pltpu.VMEM((_T, _BPW), jnp.float32),
        pltpu.SemaphoreType.DMA((2,)),
    ],
    compiler_params=pltpu.CompilerParams(use_tc_tiling_on_sc=False),
)
def _sc_gather_sum(clfT, lr_km, xp_hbm, acc_hbm, idx_v, vbuf, acc_v, sems):
    wid = lax.axis_index("s") * _NC + lax.axis_index("c")
    b0 = wid * _BPW
    for f in range(_F):
        pltpu.sync_copy(xp_hbm.at[pl.ds(f * _B + b0, _BPW)], idx_v.at[f])

    zero16 = jnp.zeros((16,), jnp.float32)

    def issue(ch, f, p):
        idx = idx_v.at[f].at[pl.ds(ch * _CB, _CB)]
        for c in range(_K):
            pltpu.async_copy(clfT.at[c].at[idx], vbuf.at[p, c], sems.at[p])
        for k in range(_K):
            pltpu.async_copy(lr_km.at[pl.ds(k * _V, _V)].at[idx],
                             vbuf.at[p, _K + k], sems.at[p])

    def wait(p):
        for t in range(_T):
            pltpu.make_async_copy(clfT.at[0].at[pl.ds(0, _CB)],
                                  vbuf.at[p, t], sems.at[p]).wait()

    def accum(ch, p):
        for t in range(_T):
            for q in range(_CB // 16):
                o = ch * _CB + 16 * q
                acc_v[t, pl.ds(o, 16)] = (
                    acc_v[t, pl.ds(o, 16)] + vbuf[p, t, pl.ds(16 * q, 16)])

    for ch in range(_NCH):
        for t in range(_T):
            for q in range(_CB // 16):
                acc_v[t, pl.ds(ch * _CB + 16 * q, 16)] = zero16

        issue(ch, 0, 0)
        issue(ch, 1, 1)

        @pl.loop(0, _F // 2)
        def fh(h):
            f0 = h * 2
            wait(0)
            accum(ch, 0)

            @pl.when(h < _F // 2 - 1)
            def _():
                issue(ch, f0 + 2, 0)

            wait(1)
            accum(ch, 1)

            @pl.when(h < _F // 2 - 1)
            def _():
                issue(ch, f0 + 3, 1)

    pltpu.sync_copy(acc_v, acc_hbm.at[wid])


def _combine_body(acc_ref, bias_ref, o_ref):
    a = acc_ref[0]
    clf_l = a[:_K, :]
    m = jnp.max(clf_l, axis=0, keepdims=True)
    e = jnp.exp(clf_l - m)
    clf = e / jnp.sum(e, axis=0, keepdims=True)
    z = a[_K:_T, :] + bias_ref[...]
    lr = 1.0 / (1.0 + jnp.exp(-z))
    o_ref[0] = jnp.sum(clf * lr, axis=0, keepdims=True)


def kernel(x, W_clf, W_lr, bias):
    clfT = W_clf.T
    lr_km = W_lr.reshape(_K * _V)
    xp = x.T.reshape(_F * _B)
    acc = _sc_gather_sum(clfT, lr_km, xp)
    out2d = pl.pallas_call(
        _combine_body,
        grid=(_NW,),
        in_specs=[
            pl.BlockSpec((1, _T, _BPW), lambda w: (w, 0, 0)),
            pl.BlockSpec((_K, 1), lambda w: (0, 0)),
        ],
        out_specs=pl.BlockSpec((1, 1, _BPW), lambda w: (w, 0, 0)),
        out_shape=jax.ShapeDtypeStruct((_NW, 1, _BPW), jnp.float32),
    )(acc, bias.reshape(_K, 1))
    return out2d.reshape(_B, 1)


# speedup vs baseline: 1.8428x; 1.8428x over previous
import functools

import jax
import jax.numpy as jnp
from jax import lax
from jax.experimental import pallas as pl
from jax.experimental.pallas import tpu as pltpu
from jax.experimental.pallas import tpu_sc as plsc

_V = 1000000
_B = 16384
_F = 26
_K = 5
_D = 16

_NC = 2
_NS = 16
_NW = _NC * _NS
_BPW = _B // _NW
_IPW = _BPW * _F
_GB = 4
_GI = _GB * _F
_NG = _BPW // _GB
_NBUF = 4

_mesh = plsc.VectorSubcoreMesh(core_axis_name="c", subcore_axis_name="s")


@functools.partial(
    pl.kernel,
    out_type=jax.ShapeDtypeStruct((_B, 2, _D), jnp.bfloat16),
    mesh=_mesh,
    scratch_types=[
        pltpu.VMEM((_IPW,), jnp.int32),
        pltpu.VMEM((_NBUF, _GI, _D), jnp.bfloat16),
        pltpu.VMEM((_BPW, 2, _D), jnp.bfloat16),
        pltpu.SemaphoreType.DMA((_NBUF,)),
    ],
    compiler_params=pltpu.CompilerParams(use_tc_tiling_on_sc=False),
)
def _sc_gather_sum(tbl_hbm, idx_hbm, acc_hbm, idx_v, buf_v, out_v, sems):
    wid = lax.axis_index("s") * _NC + lax.axis_index("c")
    pltpu.sync_copy(idx_hbm.at[pl.ds(wid * _IPW, _IPW)], idx_v)

    def start(g, d):
        pltpu.async_copy(tbl_hbm.at[idx_v.at[pl.ds(g * _GI, _GI)]],
                         buf_v.at[d], sems.at[d])

    def wait(d):
        pltpu.make_async_copy(
            tbl_hbm.at[pl.ds(0, _GI)], buf_v.at[d], sems.at[d]).wait()

    for d in range(_NBUF):
        start(d, d)

    @pl.loop(0, _NG // _NBUF)
    def outer(t):
        for d in range(_NBUF):
            g = t * _NBUF + d
            wait(d)
            for bb in range(_GB):
                r0 = bb * _F
                v = buf_v[d, pl.ds(r0, 2), :]
                for f in range(2, _F, 2):
                    v = v + buf_v[d, pl.ds(r0 + f, 2), :]
                out_v[g * _GB + bb, :, :] = v

            @pl.when(t < _NG // _NBUF - 1)
            def _():
                start(g + _NBUF, d)

    pltpu.sync_copy(out_v, acc_hbm.at[pl.ds(wid * _BPW, _BPW)])


def _combine_body(acc_ref, bias_ref, o_ref):
    a = acc_ref[...].astype(jnp.float32)
    s = a[:, 0, :] + a[:, 1, :]
    clf_l = s[:, :_K]
    m = jnp.max(clf_l, axis=1, keepdims=True)
    e = jnp.exp(clf_l - m)
    clf = e / jnp.sum(e, axis=1, keepdims=True)
    z = s[:, _K:2 * _K] + bias_ref[...]
    lr = 1.0 / (1.0 + jnp.exp(-z))
    o_ref[...] = jnp.sum(clf * lr, axis=1, keepdims=True)


def kernel(x, W_clf, W_lr, bias):
    lrT = W_lr[:, :, 0].T
    tbl = jnp.concatenate(
        [W_clf, lrT, jnp.zeros((_V, _D - 2 * _K), jnp.float32)],
        axis=1).astype(jnp.bfloat16)
    xf = x.reshape(_B * _F)
    acc = _sc_gather_sum(tbl, xf)
    out = pl.pallas_call(
        _combine_body,
        grid=(4,),
        in_specs=[
            pl.BlockSpec((_B // 4, 2, _D), lambda i: (i, 0, 0)),
            pl.BlockSpec((1, _K), lambda i: (0, 0)),
        ],
        out_specs=pl.BlockSpec((_B // 4, 1), lambda i: (i, 0)),
        out_shape=jax.ShapeDtypeStruct((_B, 1), jnp.float32),
    )(acc, bias.reshape(1, _K))
    return out


# speedup vs baseline: 1.9318x; 1.0483x over previous
import functools

import jax
import jax.numpy as jnp
from jax import lax
from jax.experimental import pallas as pl
from jax.experimental.pallas import tpu as pltpu
from jax.experimental.pallas import tpu_sc as plsc

_V = 1000000
_B = 16384
_F = 26
_K = 5
_D = 16

_NC = 2
_NS = 16
_NW = _NC * _NS
_BPW = _B // _NW
_IPW = _BPW * _F
_GB = 4
_GI = _GB * _F
_NG = _BPW // _GB
_NBUF = 4

_mesh = plsc.VectorSubcoreMesh(core_axis_name="c", subcore_axis_name="s")


@functools.partial(
    pl.kernel,
    out_type=jax.ShapeDtypeStruct((_B, _D), jnp.float32),
    mesh=_mesh,
    scratch_types=[
        pltpu.VMEM((_IPW,), jnp.int32),
        pltpu.VMEM((_NBUF, _GI, _D), jnp.float32),
        pltpu.VMEM((_BPW, _D), jnp.float32),
        pltpu.SemaphoreType.DMA((_NBUF,)),
    ],
    compiler_params=pltpu.CompilerParams(use_tc_tiling_on_sc=False),
)
def _sc_gather_sum(tbl_hbm, idx_hbm, acc_hbm, idx_v, buf_v, out_v, sems):
    wid = lax.axis_index("s") * _NC + lax.axis_index("c")
    pltpu.sync_copy(idx_hbm.at[pl.ds(wid * _IPW, _IPW)], idx_v)

    def start(g, d):
        pltpu.async_copy(tbl_hbm.at[idx_v.at[pl.ds(g * _GI, _GI)]],
                         buf_v.at[d], sems.at[d])

    def wait(d):
        pltpu.make_async_copy(
            tbl_hbm.at[pl.ds(0, _GI)], buf_v.at[d], sems.at[d]).wait()

    for d in range(_NBUF):
        start(d, d)

    @pl.loop(0, _NG // _NBUF)
    def outer(t):
        for d in range(_NBUF):
            g = t * _NBUF + d
            wait(d)
            for bb in range(_GB):
                r0 = bb * _F
                v = buf_v[d, r0, :]
                for f in range(1, _F):
                    v = v + buf_v[d, r0 + f, :]
                out_v[g * _GB + bb, :] = v

            @pl.when(t < _NG // _NBUF - 1)
            def _():
                start(g + _NBUF, d)

    pltpu.sync_copy(out_v, acc_hbm.at[pl.ds(wid * _BPW, _BPW)])


def _combine_body(acc_ref, bias_ref, o_ref):
    s = acc_ref[...]
    clf_l = s[:, :_K]
    m = jnp.max(clf_l, axis=1, keepdims=True)
    e = jnp.exp(clf_l - m)
    clf = e / jnp.sum(e, axis=1, keepdims=True)
    z = s[:, _K:2 * _K] + bias_ref[...]
    lr = 1.0 / (1.0 + jnp.exp(-z))
    o_ref[...] = jnp.sum(clf * lr, axis=1, keepdims=True)


def kernel(x, W_clf, W_lr, bias):
    lrT = W_lr[:, :, 0].T
    tbl = jnp.concatenate(
        [W_clf, lrT, jnp.zeros((_V, _D - 2 * _K), jnp.float32)], axis=1)
    xf = x.reshape(_B * _F)
    acc = _sc_gather_sum(tbl, xf)
    out = pl.pallas_call(
        _combine_body,
        grid=(4,),
        in_specs=[
            pl.BlockSpec((_B // 4, _D), lambda i: (i, 0)),
            pl.BlockSpec((1, _K), lambda i: (0, 0)),
        ],
        out_specs=pl.BlockSpec((_B // 4, 1), lambda i: (i, 0)),
        out_shape=jax.ShapeDtypeStruct((_B, 1), jnp.float32),
    )(acc, bias.reshape(1, _K))
    return out
